# baseline (speedup 1.0000x reference)
import functools

import jax
import jax.numpy as jnp
from jax.experimental import pallas as pl
from jax.experimental.pallas import tpu as pltpu


def _nms_body(nb, bl, c, n, nplanes, hidt_ref, w_ref, b_ref, x1_ref, y1_ref,
              x2_ref, y2_ref, out_ref, commit_ref, p_ref, rm_ref, ra_ref,
              aff_ref, supp_ref):
    f32 = jnp.float32
    big = jnp.int32(1 << 30)

    def init_blk(i, carry):
        hs = hidt_ref[:, pl.ds(i * bl, bl)]
        d = jnp.dot(w_ref[...], hs, preferred_element_type=f32) + b_ref[...]
        out_ref[i] = d
        ciota = jax.lax.broadcasted_iota(jnp.int32, d.shape, 0)
        dm = jnp.where(ciota >= c, -1e30, d)
        p = jax.nn.softmax(dm, axis=0)
        p = jnp.where(ciota == 0, 0.0, p)
        row = i * bl + jax.lax.broadcasted_iota(jnp.int32, d.shape, 1)
        p = jnp.where(row >= n, -3.0, p)
        p_ref[i] = p
        rmb = jnp.max(p, axis=0)
        rm_ref[i] = rmb
        ra_ref[i] = jnp.min(jnp.where(p == rmb[None, :], ciota, big), axis=0)
        return carry

    jax.lax.fori_loop(0, nb, init_blk, 0)
    commit_ref[...] = jnp.zeros((nb, bl), jnp.int32)
    supp_ref[...] = jnp.zeros((nplanes, nb, bl), jnp.int32)

    row2d = (jax.lax.broadcasted_iota(jnp.int32, (nb, bl), 0) * bl
             + jax.lax.broadcasted_iota(jnp.int32, (nb, bl), 1))

    def recompute_one(r2):
        i2 = r2 // bl
        j2 = r2 % bl
        pblk = p_ref[i2]
        ciota2 = jax.lax.broadcasted_iota(jnp.int32, pblk.shape, 0)
        supw = jnp.zeros(pblk.shape, jnp.int32)
        for pp in range(nplanes):
            supw = jnp.where(ciota2 >> 5 == pp, supp_ref[pp, i2][None, :],
                             supw)
        supbit = (supw >> (ciota2 & 31)) & 1
        eff = jnp.where(supbit == 1, 0.0, pblk)
        liota = jax.lax.broadcasted_iota(jnp.int32, pblk.shape, 1)
        effj = jnp.where(liota == j2, eff, -9.0)
        nm = jnp.max(effj)
        na = jnp.min(jnp.where(effj == nm, ciota2, big))
        lsel2 = jax.lax.broadcasted_iota(jnp.int32, (1, bl), 1) == j2
        rm_ref[pl.ds(i2, 1), :] = jnp.where(lsel2, nm, rm_ref[pl.ds(i2, 1), :])
        ra_ref[pl.ds(i2, 1), :] = jnp.where(lsel2, na, ra_ref[pl.ds(i2, 1), :])
        aff_ref[pl.ds(i2, 1), :] = jnp.where(lsel2, 0,
                                             aff_ref[pl.ds(i2, 1), :])
        return jnp.min(jnp.where(aff_ref[...] != 0, row2d, big))

    def step(t, carry):
        rm = rm_ref[...]
        r = jnp.argmax(rm.reshape(nb * bl))
        bi = r // bl
        bj = r % bl
        boh = row2d == r
        liota = jax.lax.broadcasted_iota(jnp.int32, (1, bl), 1)
        lsel = liota == bj

        def pick_f(ref, fill):
            return jnp.max(jnp.where(lsel, ref[pl.ds(bi, 1), :], fill))

        cls = jnp.max(jnp.where(lsel, ra_ref[pl.ds(bi, 1), :], 0))
        commit_ref[...] = jnp.where(boh, cls, commit_ref[...])
        x1 = x1_ref[...]
        y1 = y1_ref[...]
        x2 = x2_ref[...]
        y2 = y2_ref[...]
        zero = jnp.zeros((), f32)
        cx1 = pick_f(x1_ref, -1e30)
        cy1 = pick_f(y1_ref, -1e30)
        cx2 = pick_f(x2_ref, -1e30)
        cy2 = pick_f(y2_ref, -1e30)
        ix = jnp.clip(jnp.minimum(x2, cx2) - jnp.maximum(x1, cx1), zero, None)
        iy = jnp.clip(jnp.minimum(y2, cy2) - jnp.maximum(y1, cy1), zero, None)
        inter = ix * iy
        area = (x2 - x1) * (y2 - y1)
        carea = (cx2 - cx1) * (cy2 - cy1)
        union = jnp.maximum(area + carea - inter, 1e-8)
        ov = inter / union >= 0.5
        bit = jnp.int32(1) << (cls & 31)
        plane = cls >> 5
        sp = supp_ref[plane]
        supp_ref[plane] = jnp.where(ov, sp | bit, sp)
        rm_ref[...] = jnp.where(boh, -1.0, rm)
        affected = ov & (ra_ref[...] == cls) & (rm >= 0.0) & ~boh
        aff_ref[...] = affected.astype(jnp.int32)
        r2c0 = jnp.min(jnp.where(affected, row2d, big))
        jax.lax.while_loop(lambda r2c: r2c < big, recompute_one, r2c0)
        return carry

    jax.lax.fori_loop(0, n, step, 0)


def kernel(hidden, W_out, b_out, boxes):
    n, h = hidden.shape
    c = W_out.shape[0]
    bl = 128
    nb = (n + bl - 1) // bl
    npad = nb * bl
    cpad = ((c + 7) // 8) * 8
    nplanes = (cpad + 31) // 32

    hidt = jnp.zeros((h, npad), jnp.float32).at[:, :n].set(hidden.T)
    w = jnp.zeros((cpad, h), jnp.float32).at[:c].set(W_out)
    b = jnp.zeros((cpad, 1), jnp.float32).at[:c, 0].set(b_out)
    b = jnp.broadcast_to(b, (cpad, bl))
    bx = jnp.full((npad, 4), 2.0e9, jnp.float32).at[:n].set(boxes)
    x1 = bx[:, 0].reshape(nb, bl)
    y1 = bx[:, 1].reshape(nb, bl)
    x2 = bx[:, 2].reshape(nb, bl)
    y2 = bx[:, 3].reshape(nb, bl)

    body = functools.partial(_nms_body, nb, bl, c, n, nplanes)
    out_dists, commit = pl.pallas_call(
        body,
        out_shape=[
            jax.ShapeDtypeStruct((nb, cpad, bl), jnp.float32),
            jax.ShapeDtypeStruct((nb, bl), jnp.int32),
        ],
        scratch_shapes=[
            pltpu.VMEM((nb, cpad, bl), jnp.float32),
            pltpu.VMEM((nb, bl), jnp.float32),
            pltpu.VMEM((nb, bl), jnp.int32),
            pltpu.VMEM((nb, bl), jnp.int32),
            pltpu.VMEM((nplanes, nb, bl), jnp.int32),
        ],
    )(hidt, w, b, x1, y1, x2, y2)

    out_dists = jnp.transpose(out_dists, (0, 2, 1)).reshape(npad, cpad)[:n, :c]
    commitments = commit.reshape(npad)[:n]
    return out_dists, commitments


# speedup vs baseline: 1.2184x; 1.2184x over previous
import functools

import jax
import jax.numpy as jnp
from jax.experimental import pallas as pl
from jax.experimental.pallas import tpu as pltpu


def _nms_body(nb, bl, c, n, nplanes, hidt_ref, w_ref, b_ref, x1_ref, y1_ref,
              x2_ref, y2_ref, out_ref, commit_ref, p_ref, rm_ref, ra_ref,
              aff_ref, supp_ref):
    f32 = jnp.float32
    big = jnp.int32(1 << 30)

    def init_blk(i, carry):
        hs = hidt_ref[:, pl.ds(i * bl, bl)]
        d = jnp.dot(w_ref[...], hs, preferred_element_type=f32) + b_ref[...]
        out_ref[i] = d
        ciota = jax.lax.broadcasted_iota(jnp.int32, d.shape, 0)
        dm = jnp.where(ciota >= c, -1e30, d)
        p = jax.nn.softmax(dm, axis=0)
        p = jnp.where(ciota == 0, 0.0, p)
        row = i * bl + jax.lax.broadcasted_iota(jnp.int32, d.shape, 1)
        p = jnp.where(row >= n, -3.0, p)
        p_ref[i] = p
        rmb = jnp.max(p, axis=0)
        rm_ref[i] = rmb
        ra_ref[i] = jnp.min(jnp.where(p == rmb[None, :], ciota, big), axis=0)
        return carry

    jax.lax.fori_loop(0, nb, init_blk, 0)
    commit_ref[...] = jnp.zeros((nb, bl), jnp.int32)
    supp_ref[...] = jnp.zeros((nplanes, nb, bl), jnp.int32)

    row2d = (jax.lax.broadcasted_iota(jnp.int32, (nb, bl), 0) * bl
             + jax.lax.broadcasted_iota(jnp.int32, (nb, bl), 1))

    def recompute_one(carry):
        aff = aff_ref[...]
        r2 = jnp.min(jnp.where(aff != 0, row2d, big))
        i2 = r2 // bl
        j2 = r2 % bl
        pblk = p_ref[i2]
        ciota2 = jax.lax.broadcasted_iota(jnp.int32, pblk.shape, 0)
        supw = jnp.zeros(pblk.shape, jnp.int32)
        for pp in range(nplanes):
            supw = jnp.where(ciota2 >> 5 == pp, supp_ref[pp, i2][None, :],
                             supw)
        supbit = (supw >> (ciota2 & 31)) & 1
        eff = jnp.where(supbit == 1, 0.0, pblk)
        liota = jax.lax.broadcasted_iota(jnp.int32, pblk.shape, 1)
        effj = jnp.where(liota == j2, eff, -9.0)
        nm = jnp.max(effj)
        na = jnp.min(jnp.where(effj == nm, ciota2, big))
        boh2 = row2d == r2
        rm_ref[...] = jnp.where(boh2, nm, rm_ref[...])
        ra_ref[...] = jnp.where(boh2, na, ra_ref[...])
        aff_ref[...] = jnp.where(boh2, 0, aff)
        return jnp.any((aff != 0) & ~boh2)

    x1 = x1_ref[...]
    y1 = y1_ref[...]
    x2 = x2_ref[...]
    y2 = y2_ref[...]
    area = (x2 - x1) * (y2 - y1)
    zero = jnp.zeros((), f32)

    def overlap(r):
        boh = row2d == r
        cx1 = jnp.max(jnp.where(boh, x1, -1e30))
        cy1 = jnp.max(jnp.where(boh, y1, -1e30))
        cx2 = jnp.max(jnp.where(boh, x2, -1e30))
        cy2 = jnp.max(jnp.where(boh, y2, -1e30))
        ix = jnp.clip(jnp.minimum(x2, cx2) - jnp.maximum(x1, cx1), zero, None)
        iy = jnp.clip(jnp.minimum(y2, cy2) - jnp.maximum(y1, cy1), zero, None)
        inter = ix * iy
        carea = (cx2 - cx1) * (cy2 - cy1)
        union = jnp.maximum(area + carea - inter, 1e-8)
        return inter / union >= 0.5

    def apply_pick(r, cls, ra, rm_pre, ov):
        boh = row2d == r
        commit_ref[...] = jnp.where(boh, cls, commit_ref[...])
        bit = jnp.int32(1) << (cls & 31)
        plane = cls >> 5
        sp = supp_ref[plane]
        supp_ref[plane] = jnp.where(ov, sp | bit, sp)
        rm_ref[...] = jnp.where(boh, -1.0, rm_ref[...])
        return ov & (ra == cls) & (rm_pre >= 0.0) & ~boh

    def step(carry):
        commits = carry
        rm = rm_ref[...]
        ra = ra_ref[...]
        flat = rm.reshape(nb * bl)
        r1 = jnp.argmax(flat)
        boh1 = row2d == r1
        rm_m = jnp.where(boh1, -4.0, rm)
        r2 = jnp.argmax(rm_m.reshape(nb * bl))
        boh2 = row2d == r2
        cls1 = jnp.max(jnp.where(boh1, ra, 0))
        cls2 = jnp.max(jnp.where(boh2, ra, 0))
        v2 = jnp.max(rm_m)
        ov1 = overlap(r1)
        ov1_at2 = jnp.any(ov1 & boh2)

        aff1 = apply_pick(r1, cls1, ra, rm, ov1)
        valid2 = ((commits + 1 < n) & (v2 >= 0.0)
                  & ~(ov1_at2 & (cls2 == cls1)))

        @pl.when(valid2)
        def _():
            aff2 = apply_pick(r2, cls2, ra, rm, overlap(r2))
            aff_ref[...] = ((aff1 | aff2) & ~boh1).astype(jnp.int32)

        @pl.when(~valid2)
        def _():
            aff_ref[...] = aff1.astype(jnp.int32)

        jax.lax.while_loop(lambda more: more, recompute_one,
                           jnp.any(aff_ref[...] != 0))
        return commits + jnp.where(valid2, 2, 1)

    jax.lax.while_loop(lambda commits: commits < n, step, jnp.int32(0))


def kernel(hidden, W_out, b_out, boxes):
    n, h = hidden.shape
    c = W_out.shape[0]
    bl = 128
    nb = (n + bl - 1) // bl
    npad = nb * bl
    cpad = ((c + 7) // 8) * 8
    nplanes = (cpad + 31) // 32

    hidt = jnp.zeros((h, npad), jnp.float32).at[:, :n].set(hidden.T)
    w = jnp.zeros((cpad, h), jnp.float32).at[:c].set(W_out)
    b = jnp.zeros((cpad, 1), jnp.float32).at[:c, 0].set(b_out)
    b = jnp.broadcast_to(b, (cpad, bl))
    bx = jnp.full((npad, 4), 2.0e9, jnp.float32).at[:n].set(boxes)
    x1 = bx[:, 0].reshape(nb, bl)
    y1 = bx[:, 1].reshape(nb, bl)
    x2 = bx[:, 2].reshape(nb, bl)
    y2 = bx[:, 3].reshape(nb, bl)

    body = functools.partial(_nms_body, nb, bl, c, n, nplanes)
    out_dists, commit = pl.pallas_call(
        body,
        out_shape=[
            jax.ShapeDtypeStruct((nb, cpad, bl), jnp.float32),
            jax.ShapeDtypeStruct((nb, bl), jnp.int32),
        ],
        scratch_shapes=[
            pltpu.VMEM((nb, cpad, bl), jnp.float32),
            pltpu.VMEM((nb, bl), jnp.float32),
            pltpu.VMEM((nb, bl), jnp.int32),
            pltpu.VMEM((nb, bl), jnp.int32),
            pltpu.VMEM((nplanes, nb, bl), jnp.int32),
        ],
    )(hidt, w, b, x1, y1, x2, y2)

    out_dists = jnp.transpose(out_dists, (0, 2, 1)).reshape(npad, cpad)[:n, :c]
    commitments = commit.reshape(npad)[:n]
    return out_dists, commitments
